# baseline (speedup 1.0000x reference)
import jax
import jax.numpy as jnp
from jax import lax
from jax.experimental import pallas as pl
from jax.experimental.pallas import tpu as pltpu
from jax.experimental.pallas import tpu_sc as plsc

VOCAB = 1000000
D = 32
BATCH = 4096
HIST = 50
LANES = 16

NUM_CORES = 2
NUM_SUBCORES = 16
NW = NUM_CORES * NUM_SUBCORES
ROWS_PER_W = BATCH // NW
CHUNK_ROWS = 16
NCHUNKS = ROWS_PER_W // CHUNK_ROWS
CHUNK_PAIRS = CHUNK_ROWS * HIST
WPAD = 64

_BCAST_DNUMS = lax.GatherDimensionNumbers(
    offset_dims=(), collapsed_slice_dims=(0,), start_index_map=(0,))


def _lane_bcast(vec, k):
    idx = jnp.full((LANES, 1), k, jnp.int32)
    return lax.gather(vec, idx, _BCAST_DNUMS, (1,),
                      mode=lax.GatherScatterMode.PROMISE_IN_BOUNDS)


def _nbow_kernel(idx_hbm, table_hbm, tw_hbm, out_hbm,
                 idx_v, rows_v, w_v, w_pad, out_v, sem_w, sem_r):
    wid = lax.axis_index("s") * NUM_CORES + lax.axis_index("c")

    for i in range(CHUNK_ROWS):
        w_pad[pl.ds(i * WPAD + 48, LANES)] = jnp.zeros((LANES,), jnp.float32)

    def chunk_body(c, carry):
        base_pair = (wid * ROWS_PER_W + c * CHUNK_ROWS) * HIST
        base_row = wid * ROWS_PER_W + c * CHUNK_ROWS

        pltpu.sync_copy(idx_hbm.at[pl.ds(base_pair, CHUNK_PAIRS)], idx_v)
        cp_w = pltpu.async_copy(tw_hbm.at[idx_v], w_v, sem_w)
        cp_r = pltpu.async_copy(table_hbm.at[idx_v], rows_v, sem_r)
        cp_w.wait()
        cp_r.wait()

        def mask_body(g, carry2):
            pv = g * LANES + lax.iota(jnp.int32, LANES)
            iv = idx_v[pl.ds(g * LANES, LANES)]
            tw16 = w_v[pl.ds(g * LANES, LANES)]
            w = jnp.where(iv != 0, tw16, 0.0)
            hv = jnp.full((LANES,), HIST, jnp.int32)
            dest = lax.div(pv, hv) * WPAD + lax.rem(pv, hv)
            plsc.store_scatter(w_pad, [dest], w)
            return carry2
        lax.fori_loop(0, CHUNK_PAIRS // LANES, mask_body, 0, unroll=4)

        def row_body(i, carry2):
            p0 = i * HIST
            acc0 = jnp.zeros((LANES,), jnp.float32)
            acc1 = jnp.zeros((LANES,), jnp.float32)
            for g in range(4):
                w16 = w_pad[pl.ds(i * WPAD + g * LANES, LANES)]
                for k in range(LANES if g < 3 else HIST - 3 * LANES):
                    p = p0 + g * LANES + k
                    wv = _lane_bcast(w16, k)
                    acc0 = acc0 + wv * rows_v[p, 0:LANES]
                    acc1 = acc1 + wv * rows_v[p, LANES:2 * LANES]
            out_v[i, 0:LANES] = acc0
            out_v[i, LANES:2 * LANES] = acc1
            return carry2
        lax.fori_loop(0, CHUNK_ROWS, row_body, 0)

        pltpu.sync_copy(out_v, out_hbm.at[pl.ds(base_row, CHUNK_ROWS)])
        return carry

    lax.fori_loop(0, NCHUNKS, chunk_body, 0)


@jax.jit
def kernel(idxs, table, token_weights):
    idx_flat = idxs.reshape(BATCH * HIST).astype(jnp.int32)
    table_flat = jax.lax.optimization_barrier(table.reshape(VOCAB * D))
    table_lin = table_flat.reshape(VOCAB, D)
    mesh = plsc.VectorSubcoreMesh(core_axis_name="c", subcore_axis_name="s")
    f = pl.kernel(
        _nbow_kernel,
        mesh=mesh,
        compiler_params=pltpu.CompilerParams(
            use_tc_tiling_on_sc=False, needs_layout_passes=False),
        out_type=jax.ShapeDtypeStruct((BATCH, D), jnp.float32),
        scratch_types=[
            pltpu.VMEM((CHUNK_PAIRS,), jnp.int32),
            pltpu.VMEM((CHUNK_PAIRS, D), jnp.float32),
            pltpu.VMEM((CHUNK_PAIRS,), jnp.float32),
            pltpu.VMEM((CHUNK_ROWS * WPAD,), jnp.float32),
            pltpu.VMEM((CHUNK_ROWS, D), jnp.float32),
            pltpu.SemaphoreType.DMA,
            pltpu.SemaphoreType.DMA,
        ],
    )
    return f(idx_flat, table_lin, token_weights)


# speedup vs baseline: 1.7229x; 1.7229x over previous
import jax
import jax.numpy as jnp
from jax import lax
from jax.experimental import pallas as pl
from jax.experimental.pallas import tpu as pltpu
from jax.experimental.pallas import tpu_sc as plsc

VOCAB = 1000000
D = 32
BATCH = 4096
HIST = 50
LANES = 16

NUM_CORES = 2
NUM_SUBCORES = 16
NW = NUM_CORES * NUM_SUBCORES
ROWS_PER_W = BATCH // NW
CHUNK_ROWS = 16
NCHUNKS = ROWS_PER_W // CHUNK_ROWS
CHUNK_PAIRS = CHUNK_ROWS * HIST


def _nbow_kernel(idx_hbm, table_hbm, out_hbm, idx_v, rows_v, out_v, sem_r):
    wid = lax.axis_index("s") * NUM_CORES + lax.axis_index("c")

    def chunk_body(c, carry):
        base_pair = (wid * ROWS_PER_W + c * CHUNK_ROWS) * HIST
        base_row = wid * ROWS_PER_W + c * CHUNK_ROWS

        pltpu.sync_copy(idx_hbm.at[pl.ds(base_pair, CHUNK_PAIRS)], idx_v)
        cp_r = pltpu.async_copy(table_hbm.at[idx_v], rows_v, sem_r)
        cp_r.wait()

        def row_body(i, carry2):
            p0 = i * HIST
            acc0 = jnp.zeros((LANES,), jnp.float32)
            acc1 = jnp.zeros((LANES,), jnp.float32)
            acc2 = jnp.zeros((LANES,), jnp.float32)
            acc3 = jnp.zeros((LANES,), jnp.float32)
            for k in range(0, HIST, 2):
                acc0 = acc0 + rows_v[p0 + k, 0:LANES]
                acc1 = acc1 + rows_v[p0 + k, LANES:2 * LANES]
                acc2 = acc2 + rows_v[p0 + k + 1, 0:LANES]
                acc3 = acc3 + rows_v[p0 + k + 1, LANES:2 * LANES]
            out_v[i, 0:LANES] = acc0 + acc2
            out_v[i, LANES:2 * LANES] = acc1 + acc3
            return carry2
        lax.fori_loop(0, CHUNK_ROWS, row_body, 0)

        pltpu.sync_copy(out_v, out_hbm.at[pl.ds(base_row, CHUNK_ROWS)])
        return carry

    lax.fori_loop(0, NCHUNKS, chunk_body, 0)


TBLK = 16128
NFULL = VOCAB // TBLK
TTAIL = VOCAB - NFULL * TBLK
OBLK = TBLK // 4


def _relayout_kernel(tT_hbm, tw_hbm, tail_ref, twt_ref, out_hbm,
                     xv0, xv1, twv0, twv1, y2v, yv0, yv1,
                     sx0, sx1, st0, st1, sy0, sy1):
    b = pl.program_id(0)

    def in_x(w, xv, sem):
        return pltpu.make_async_copy(
            tT_hbm.at[:, pl.ds(w * TBLK, TBLK)], xv, sem)

    def in_t(w, twv, sem):
        return pltpu.make_async_copy(
            tw_hbm.at[:, pl.ds(w * TBLK, TBLK)], twv, sem)

    def out_y(w, yv, sem):
        return pltpu.make_async_copy(
            yv, out_hbm.at[pl.ds(w * OBLK, OBLK), :], sem)

    @pl.when(b == 0)
    def _prime():
        in_x(0, xv0, sx0).start()
        in_t(0, twv0, st0).start()
        in_x(1, xv1, sx1).start()
        in_t(1, twv1, st1).start()

    def step(xv, twv, yv, sx, st, sy):
        in_x(b, xv, sx).wait()
        in_t(b, twv, st).wait()
        lane = lax.broadcasted_iota(jnp.int32, (1, TBLK), 1)
        tw_eff = jnp.where((b == 0) & (lane == 0), 0.0, twv[...])
        y2v[...] = (xv[...] * tw_eff).T
        @pl.when(b >= 2)
        def _():
            out_y(b - 2, yv, sy).wait()
        yv[...] = jnp.concatenate(
            [y2v[pl.Slice(g, OBLK, 4), :] for g in range(4)], axis=1)
        out_y(b, yv, sy).start()
        @pl.when(b + 2 < NFULL)
        def _():
            in_x(b + 2, xv, sx).start()
            in_t(b + 2, twv, st).start()

    @pl.when((b < NFULL) & (lax.rem(b, 2) == 0))
    def _even():
        step(xv0, twv0, yv0, sx0, st0, sy0)

    @pl.when((b < NFULL) & (lax.rem(b, 2) == 1))
    def _odd():
        step(xv1, twv1, yv1, sx1, st1, sy1)

    @pl.when(b == NFULL)
    def _tail():
        out_y(NFULL - 2, yv0, sy0).wait()
        out_y(NFULL - 1, yv1, sy1).wait()
        y2v[:TTAIL, :] = (tail_ref[...] * twt_ref[...]).T
        yv0[: TTAIL // 4, :] = jnp.concatenate(
            [y2v[pl.Slice(g, TTAIL // 4, 4), :] for g in range(4)], axis=1)
        cout = pltpu.make_async_copy(
            yv0.at[: TTAIL // 4, :],
            out_hbm.at[pl.ds(NFULL * OBLK, TTAIL // 4), :], sy0)
        cout.start()
        cout.wait()


def _relayout_table(tT, tw2):
    tail = lax.slice(tT, (0, NFULL * TBLK), (D, VOCAB))
    twt = lax.slice(tw2, (0, NFULL * TBLK), (1, VOCAB))
    return pl.pallas_call(
        _relayout_kernel,
        grid=(NFULL + 1,),
        in_specs=[pl.BlockSpec(memory_space=pl.ANY),
                  pl.BlockSpec(memory_space=pl.ANY),
                  pl.BlockSpec((D, TTAIL), lambda b: (0, 0)),
                  pl.BlockSpec((1, TTAIL), lambda b: (0, 0))],
        out_specs=pl.BlockSpec(memory_space=pl.ANY),
        out_shape=jax.ShapeDtypeStruct((VOCAB // 4, 4 * D), jnp.float32),
        scratch_shapes=[
            pltpu.VMEM((D, TBLK), jnp.float32),
            pltpu.VMEM((D, TBLK), jnp.float32),
            pltpu.VMEM((1, TBLK), jnp.float32),
            pltpu.VMEM((1, TBLK), jnp.float32),
            pltpu.VMEM((TBLK, D), jnp.float32),
            pltpu.VMEM((OBLK, 4 * D), jnp.float32),
            pltpu.VMEM((OBLK, 4 * D), jnp.float32),
            pltpu.SemaphoreType.DMA,
            pltpu.SemaphoreType.DMA,
            pltpu.SemaphoreType.DMA,
            pltpu.SemaphoreType.DMA,
            pltpu.SemaphoreType.DMA,
            pltpu.SemaphoreType.DMA,
        ],
    )(tT, tw2, tail, twt)


@jax.jit
def kernel(idxs, table, token_weights):
    idx_flat = idxs.reshape(BATCH * HIST).astype(jnp.int32)
    tw2 = token_weights.reshape(1, VOCAB)
    table_lin = _relayout_table(table.T, tw2).reshape(VOCAB, D)
    mesh = plsc.VectorSubcoreMesh(core_axis_name="c", subcore_axis_name="s")
    f = pl.kernel(
        _nbow_kernel,
        mesh=mesh,
        compiler_params=pltpu.CompilerParams(
            use_tc_tiling_on_sc=False, needs_layout_passes=False),
        out_type=jax.ShapeDtypeStruct((BATCH, D), jnp.float32),
        scratch_types=[
            pltpu.VMEM((CHUNK_PAIRS,), jnp.int32),
            pltpu.VMEM((CHUNK_PAIRS, D), jnp.float32),
            pltpu.VMEM((CHUNK_ROWS, D), jnp.float32),
            pltpu.SemaphoreType.DMA,
        ],
    )
    return f(idx_flat, table_lin)


# speedup vs baseline: 1.7427x; 1.0115x over previous
import jax
import jax.numpy as jnp
from jax import lax
from jax.experimental import pallas as pl
from jax.experimental.pallas import tpu as pltpu
from jax.experimental.pallas import tpu_sc as plsc

VOCAB = 1000000
D = 32
BATCH = 4096
HIST = 50
LANES = 16

NUM_CORES = 2
NUM_SUBCORES = 16
NW = NUM_CORES * NUM_SUBCORES
ROWS_PER_W = BATCH // NW
CHUNK_ROWS = 16
NCHUNKS = ROWS_PER_W // CHUNK_ROWS
CHUNK_PAIRS = CHUNK_ROWS * HIST


def _nbow_kernel(idx_hbm, table_hbm, out_hbm, idx_v, rows_v, out_v, sem_r):
    wid = lax.axis_index("s") * NUM_CORES + lax.axis_index("c")

    def chunk_body(c, carry):
        base_pair = (wid * ROWS_PER_W + c * CHUNK_ROWS) * HIST
        base_row = wid * ROWS_PER_W + c * CHUNK_ROWS

        pltpu.sync_copy(idx_hbm.at[pl.ds(base_pair, CHUNK_PAIRS)], idx_v)
        cp_r = pltpu.async_copy(table_hbm.at[idx_v], rows_v, sem_r)
        cp_r.wait()

        def row_body(i, carry2):
            p0 = i * HIST
            acc0 = jnp.zeros((LANES,), jnp.float32)
            acc1 = jnp.zeros((LANES,), jnp.float32)
            acc2 = jnp.zeros((LANES,), jnp.float32)
            acc3 = jnp.zeros((LANES,), jnp.float32)
            for k in range(0, HIST, 2):
                acc0 = acc0 + rows_v[p0 + k, 0:LANES]
                acc1 = acc1 + rows_v[p0 + k, LANES:2 * LANES]
                acc2 = acc2 + rows_v[p0 + k + 1, 0:LANES]
                acc3 = acc3 + rows_v[p0 + k + 1, LANES:2 * LANES]
            out_v[i, 0:LANES] = acc0 + acc2
            out_v[i, LANES:2 * LANES] = acc1 + acc3
            return carry2
        lax.fori_loop(0, CHUNK_ROWS, row_body, 0)

        pltpu.sync_copy(out_v, out_hbm.at[pl.ds(base_row, CHUNK_ROWS)])
        return carry

    lax.fori_loop(0, NCHUNKS, chunk_body, 0)


TBLK = 16128
NFULL = VOCAB // TBLK
TTAIL = VOCAB - NFULL * TBLK
OBLK = TBLK // 4


def _relayout_main(xref, twref, oref, y2v):
    b = pl.program_id(0)
    lane = lax.broadcasted_iota(jnp.int32, (1, TBLK), 1)
    tw_eff = jnp.where((b == 0) & (lane == 0), 0.0, twref[...])
    y2v[...] = (xref[...] * tw_eff).T
    oref[...] = jnp.concatenate(
        [y2v[pl.Slice(g, OBLK, 4), :] for g in range(4)], axis=1)


def _relayout_tail(pk_in, tailT, twt, out_hbm, y2t, yv, sem):
    del pk_in
    y2t[...] = (tailT[...] * twt[...]).T
    yv[...] = jnp.concatenate(
        [y2t[pl.Slice(g, TTAIL // 4, 4), :] for g in range(4)], axis=1)
    cp = pltpu.make_async_copy(
        yv, out_hbm.at[pl.ds(NFULL * OBLK, TTAIL // 4), :], sem)
    cp.start()
    cp.wait()


def _relayout_table(tT, tw2):
    packed = pl.pallas_call(
        _relayout_main,
        grid=(NFULL,),
        in_specs=[pl.BlockSpec((D, TBLK), lambda b: (0, b)),
                  pl.BlockSpec((1, TBLK), lambda b: (0, b))],
        out_specs=pl.BlockSpec((OBLK, 4 * D), lambda b: (b, 0)),
        out_shape=jax.ShapeDtypeStruct((VOCAB // 4, 4 * D), jnp.float32),
        scratch_shapes=[pltpu.VMEM((TBLK, D), jnp.float32)],
        compiler_params=pltpu.CompilerParams(
            dimension_semantics=("parallel",)),
    )(tT, tw2)
    tail = lax.slice(tT, (0, NFULL * TBLK), (D, VOCAB))
    twt = lax.slice(tw2, (0, NFULL * TBLK), (1, VOCAB))
    return pl.pallas_call(
        _relayout_tail,
        in_specs=[pl.BlockSpec(memory_space=pl.ANY),
                  pl.BlockSpec((D, TTAIL), lambda: (0, 0)),
                  pl.BlockSpec((1, TTAIL), lambda: (0, 0))],
        out_specs=pl.BlockSpec(memory_space=pl.ANY),
        out_shape=jax.ShapeDtypeStruct((VOCAB // 4, 4 * D), jnp.float32),
        input_output_aliases={0: 0},
        scratch_shapes=[
            pltpu.VMEM((TTAIL, D), jnp.float32),
            pltpu.VMEM((TTAIL // 4, 4 * D), jnp.float32),
            pltpu.SemaphoreType.DMA,
        ],
    )(packed, tail, twt)


@jax.jit
def kernel(idxs, table, token_weights):
    idx_flat = idxs.reshape(BATCH * HIST).astype(jnp.int32)
    tw2 = token_weights.reshape(1, VOCAB)
    table_lin = _relayout_table(table.T, tw2).reshape(VOCAB, D)
    mesh = plsc.VectorSubcoreMesh(core_axis_name="c", subcore_axis_name="s")
    f = pl.kernel(
        _nbow_kernel,
        mesh=mesh,
        compiler_params=pltpu.CompilerParams(
            use_tc_tiling_on_sc=False, needs_layout_passes=False),
        out_type=jax.ShapeDtypeStruct((BATCH, D), jnp.float32),
        scratch_types=[
            pltpu.VMEM((CHUNK_PAIRS,), jnp.int32),
            pltpu.VMEM((CHUNK_PAIRS, D), jnp.float32),
            pltpu.VMEM((CHUNK_ROWS, D), jnp.float32),
            pltpu.SemaphoreType.DMA,
        ],
    )
    return f(idx_flat, table_lin)
